# Initial kernel scaffold; baseline (speedup 1.0000x reference)
import jax
import jax.numpy as jnp
from jax.experimental import pallas as pl


def kernel(step, obs, action, reward, log_prob, value, obs_buf, actions_buf, rewards_buf, log_prob_buf, values_buf):
    raise NotImplementedError("write your pallas kernel here")


# speedup vs baseline: 1.1267x; 1.1267x over previous
import jax
import jax.numpy as jnp
from jax.experimental import pallas as pl
from jax.experimental.pallas import tpu as pltpu

B = 1024
T = 200
OBS = 512
BB = 256
TB = 8


def _obs_body(step_ref, obs_blk, buf_blk, out_blk):
    step = step_ref[0]
    t0 = pl.program_id(1) * TB
    tids = t0 + jax.lax.broadcasted_iota(jnp.int32, (1, TB, 1), 1)
    out_blk[...] = jnp.where(tids == step, obs_blk[...][:, None, :], buf_blk[...])


def _small_body(step_ref, act, rew, logp, val, abuf, rbuf, lbuf, vbuf,
                aout, rout, lout, vout):
    step = step_ref[0]
    col = jax.lax.broadcasted_iota(jnp.int32, (B, T), 1)
    mask = col == step
    aout[...] = jnp.where(mask, act[...], abuf[...])
    rout[...] = jnp.where(mask, rew[...], rbuf[...])
    lout[...] = jnp.where(mask, logp[...], lbuf[...])
    colv = jax.lax.broadcasted_iota(jnp.int32, (B, T + 1), 1)
    vout[...] = jnp.where(colv == step, val[...], vbuf[...])


def kernel(step, obs, action, reward, log_prob, value,
           obs_buf, actions_buf, rewards_buf, log_prob_buf, values_buf):
    step_arr = jnp.asarray(step, dtype=jnp.int32).reshape((1,))

    new_obs = pl.pallas_call(
        _obs_body,
        grid_spec=pltpu.PrefetchScalarGridSpec(
            num_scalar_prefetch=1,
            grid=(B // BB, T // TB),
            in_specs=[
                pl.BlockSpec((BB, OBS), lambda i, j, s: (i, 0)),
                pl.BlockSpec((BB, TB, OBS), lambda i, j, s: (i, j, 0)),
            ],
            out_specs=pl.BlockSpec((BB, TB, OBS), lambda i, j, s: (i, j, 0)),
        ),
        out_shape=jax.ShapeDtypeStruct((B, T, OBS), jnp.float32),
    )(step_arr, obs, obs_buf)

    new_actions, new_rewards, new_log_prob, new_values = pl.pallas_call(
        _small_body,
        grid_spec=pltpu.PrefetchScalarGridSpec(num_scalar_prefetch=1),
        out_shape=(
            jax.ShapeDtypeStruct((B, T), jnp.int32),
            jax.ShapeDtypeStruct((B, T), jnp.float32),
            jax.ShapeDtypeStruct((B, T), jnp.float32),
            jax.ShapeDtypeStruct((B, T + 1), jnp.float32),
        ),
    )(step_arr,
      action.reshape(B, 1), reward.reshape(B, 1),
      log_prob.reshape(B, 1), value.reshape(B, 1),
      actions_buf, rewards_buf, log_prob_buf, values_buf)

    return (new_obs, new_actions, new_rewards, new_log_prob, new_values)


# speedup vs baseline: 1.1635x; 1.0327x over previous
import jax
import jax.numpy as jnp
from jax.experimental import pallas as pl
from jax.experimental.pallas import tpu as pltpu

B = 1024
T = 200
OBS = 512
BB = 512
TB = 8


def _obs_body(step_ref, obs_blk, buf_blk, out_blk):
    step = step_ref[0]
    t0 = pl.program_id(1) * TB
    tids = t0 + jax.lax.broadcasted_iota(jnp.int32, (1, TB, 1), 1)
    out_blk[...] = jnp.where(tids == step, obs_blk[...][:, None, :], buf_blk[...])


def _small_body(step_ref, act, rew, logp, val, abuf, rbuf, lbuf, vbuf,
                aout, rout, lout, vout):
    step = step_ref[0]
    col = jax.lax.broadcasted_iota(jnp.int32, (B, T), 1)
    mask = col == step
    aout[...] = jnp.where(mask, act[...], abuf[...])
    rout[...] = jnp.where(mask, rew[...], rbuf[...])
    lout[...] = jnp.where(mask, logp[...], lbuf[...])
    colv = jax.lax.broadcasted_iota(jnp.int32, (B, T + 1), 1)
    vout[...] = jnp.where(colv == step, val[...], vbuf[...])


def kernel(step, obs, action, reward, log_prob, value,
           obs_buf, actions_buf, rewards_buf, log_prob_buf, values_buf):
    step_arr = jnp.asarray(step, dtype=jnp.int32).reshape((1,))

    new_obs = pl.pallas_call(
        _obs_body,
        grid_spec=pltpu.PrefetchScalarGridSpec(
            num_scalar_prefetch=1,
            grid=(B // BB, T // TB),
            in_specs=[
                pl.BlockSpec((BB, OBS), lambda i, j, s: (i, 0)),
                pl.BlockSpec((BB, TB, OBS), lambda i, j, s: (i, j, 0)),
            ],
            out_specs=pl.BlockSpec((BB, TB, OBS), lambda i, j, s: (i, j, 0)),
        ),
        out_shape=jax.ShapeDtypeStruct((B, T, OBS), jnp.float32),
    )(step_arr, obs, obs_buf)

    new_actions, new_rewards, new_log_prob, new_values = pl.pallas_call(
        _small_body,
        grid_spec=pltpu.PrefetchScalarGridSpec(num_scalar_prefetch=1),
        out_shape=(
            jax.ShapeDtypeStruct((B, T), jnp.int32),
            jax.ShapeDtypeStruct((B, T), jnp.float32),
            jax.ShapeDtypeStruct((B, T), jnp.float32),
            jax.ShapeDtypeStruct((B, T + 1), jnp.float32),
        ),
    )(step_arr,
      action.reshape(B, 1), reward.reshape(B, 1),
      log_prob.reshape(B, 1), value.reshape(B, 1),
      actions_buf, rewards_buf, log_prob_buf, values_buf)

    return (new_obs, new_actions, new_rewards, new_log_prob, new_values)
